# baseline (speedup 1.0000x reference)
import functools

import jax
import jax.numpy as jnp
from jax import lax
from jax.experimental import pallas as pl
from jax.experimental.pallas import tpu as pltpu
from jax.experimental.pallas import tpu_sc as plsc

_N = 10000
_E = 320000
_H = 128
_NH = 8
_L = 6
_G = 8

_CH = 64
_NCHUNK = _E // _CH
_NC = 2
_NS = 16
_NW = _NC * _NS
_NPAD = _N
_NPS = _NPAD // _NS

_f32 = jnp.float32


def _sc_edge_call(xp, asad, ae, li, src2, dst2, zm, zd):
    mesh = plsc.VectorSubcoreMesh(core_axis_name="c", subcore_axis_name="s")

    @functools.partial(
        pl.kernel,
        out_type=(
            jax.ShapeDtypeStruct((_NC, _NPAD, _H), _f32),
            jax.ShapeDtypeStruct((_NC, _NPAD, 16), _f32),
        ),
        mesh=mesh,
        compiler_params=pltpu.CompilerParams(use_tc_tiling_on_sc=False),
        scratch_types=[
            pltpu.VMEM_SHARED((_NPAD, _H), _f32),
            pltpu.VMEM_SHARED((_NPAD, 16), _f32),
            pltpu.VMEM((3, _CH), jnp.int32),
            pltpu.VMEM((6, _CH), jnp.int32),
            pltpu.VMEM((2, _CH, 16), _f32),
            pltpu.VMEM((2, _CH, 16), _f32),
            pltpu.VMEM((2, _CH // 16, 128), _f32),
            pltpu.VMEM((2, _CH, _H // 2), jnp.int32),
            pltpu.VMEM((3, _CH, _H), _f32),
            pltpu.VMEM((2, _CH, 16), _f32),
        ] + [pltpu.SemaphoreType.DMA] * 16,
    )
    def k(xp_h, asad_h, ae_h, src_h, dst_h, zm_h, zd_h, om_h, od_h,
          accm, accd, srcv, dstv, asv, adv, aev, xpv, wbuf, exv, *sems):
        s_as = sems[0:2]
        s_ad = sems[2:4]
        s_ae = sems[4:6]
        s_xp = sems[6:8]
        s_sm = sems[8:11]
        s_sd = sems[11:13]
        s_ix = sems[13:16]
        c = lax.axis_index("c")
        s = lax.axis_index("s")
        w = c * _NS + s

        r0 = s * _NPS
        pltpu.sync_copy(zm_h.at[pl.ds(r0, _NPS)], accm.at[pl.ds(r0, _NPS)])
        pltpu.sync_copy(zd_h.at[pl.ds(r0, _NPS)], accd.at[pl.ds(r0, _NPS)])
        plsc.subcore_barrier()

        ii = lax.iota(jnp.int32, 16)
        perm8 = ii ^ 8

        nfull = _NCHUNK // _NW
        rem = _NCHUNK - nfull * _NW
        n_my = nfull + jnp.where(w < rem, 1, 0)

        def issue_idx(kk, i3, i6):
            ci = w + kk * _NW
            pltpu.async_copy(src_h.at[ci], srcv.at[i3], s_ix[i3])
            pltpu.async_copy(dst_h.at[ci], dstv.at[i6], s_ix[i3])

        def wait_idx(i3, i6):
            pltpu.make_async_copy(src_h.at[0], srcv.at[i3], s_ix[i3]).wait()
            pltpu.make_async_copy(dst_h.at[0], dstv.at[i6], s_ix[i3]).wait()

        def issue_gathers(kk, i2, i3, i6):
            ci = w + kk * _NW
            pltpu.async_copy(asad_h.at[srcv.at[i3]], asv.at[i2], s_as[i2])
            pltpu.async_copy(asad_h.at[dstv.at[i6]], adv.at[i2], s_ad[i2])
            pltpu.async_copy(xp_h.at[srcv.at[i3]], xpv.at[i2], s_xp[i2])
            pltpu.async_copy(
                ae_h.at[li, pl.ds(ci * (_CH // 16), _CH // 16)],
                aev.at[i2], s_ae[i2])

        def wait_gathers(i2, i3):
            pltpu.make_async_copy(asad_h.at[srcv.at[i3]], asv.at[i2],
                                  s_as[i2]).wait()
            pltpu.make_async_copy(asad_h.at[srcv.at[i3]], adv.at[i2],
                                  s_ad[i2]).wait()
            pltpu.make_async_copy(xp_h.at[srcv.at[i3]], xpv.at[i2],
                                  s_xp[i2]).wait()
            pltpu.make_async_copy(ae_h.at[0, pl.ds(0, _CH // 16)],
                                  aev.at[i2], s_ae[i2]).wait()

        def issue_scatters(i2, i3, i6):
            pltpu.async_copy(wbuf.at[i3], accm.at[dstv.at[i6]], s_sm[i3],
                             add=True)
            pltpu.async_copy(exv.at[i2], accd.at[dstv.at[i6]], s_sd[i2],
                             add=True)

        def wait_scatter_m(i3):
            pltpu.make_async_copy(wbuf.at[i3], accm.at[dstv.at[0]],
                                  s_sm[i3]).wait()

        def wait_scatter_d(i2):
            pltpu.make_async_copy(exv.at[i2], accd.at[dstv.at[0]],
                                  s_sd[i2]).wait()

        def compute(i2, i3):
            def edge_body(j, carry2):
                aerow = aev[i2, j // 8, pl.ds((j % 8) * 16, 16)]
                for t in range(2):
                    e = j * 2 + t
                    va = asv[i2, e]
                    vd = adv[i2, e]
                    vae = aerow if t == 0 else jnp.take(aerow, perm8)
                    alpha = va + jnp.take(vd, perm8) + vae
                    alpha = jnp.where(alpha > 0, alpha, alpha * 0.2)
                    ex = jnp.exp(alpha)
                    exv[i2, e] = ex
                    for g in range(4):
                        u = xpv[i2, e, pl.ds(16 * g, 16)]
                        xa = lax.bitcast_convert_type(u << 16, _f32)
                        xc = lax.bitcast_convert_type(
                            jnp.bitwise_and(u, jnp.int32(-65536)), _f32)
                        ea = jnp.take(ex, jnp.full((16,), 2 * g, jnp.int32))
                        ec = jnp.take(ex, jnp.full((16,), 2 * g + 1,
                                                   jnp.int32))
                        wbuf[i3, e, pl.ds(32 * g, 16)] = xa * ea
                        wbuf[i3, e, pl.ds(32 * g + 16, 16)] = xc * ec
                return carry2

            lax.fori_loop(0, _CH // 2, edge_body, 0)

        issue_idx(0, 0, 0)
        issue_idx(1, 1, 1)
        wait_idx(0, 0)
        issue_gathers(0, 0, 0, 0)

        def step_body(g, carry):
            for bb in range(6):
                kk = g * 6 + bb
                u2 = bb % 2
                u3 = bb % 3
                u6 = bb
                q2 = (bb + 1) % 2
                q3 = (bb + 1) % 3
                q6 = (bb + 1) % 6

                @pl.when(kk < n_my)
                def _():
                    @pl.when(kk + 2 < n_my)
                    def _():
                        issue_idx(kk + 2, (bb + 2) % 3, (bb + 2) % 6)

                    wait_gathers(u2, u3)

                    @pl.when(kk + 1 < n_my)
                    def _():
                        @pl.when(kk >= 2)
                        def _():
                            wait_scatter_m(q3)
                        wait_idx(q3, q6)
                        issue_gathers(kk + 1, q2, q3, q6)

                    @pl.when(kk >= 2)
                    def _():
                        wait_scatter_d(u2)

                    compute(u2, u3)
                    issue_scatters(u2, u3, u6)
            return carry

        lax.fori_loop(0, (n_my + 5) // 6, step_body, 0)
        wait_scatter_m(0)
        wait_scatter_m(1)
        wait_scatter_m(2)
        wait_scatter_d(0)
        wait_scatter_d(1)

        plsc.subcore_barrier()
        pltpu.sync_copy(accm.at[pl.ds(r0, _NPS)], om_h.at[c, pl.ds(r0, _NPS)])
        pltpu.sync_copy(accd.at[pl.ds(r0, _NPS)], od_h.at[c, pl.ds(r0, _NPS)])

    return k(xp, asad, ae, src2, dst2, zm, zd)


_BN = 1000


def _gelu(t):
    return t * 0.5 * (1.0 + lax.erf(t * 0.7071067811865476))


def _pack_xp(t, wc):
    xq = jnp.dot(t, wc, preferred_element_type=_f32)
    u = lax.bitcast_convert_type(xq, jnp.uint32)
    r = (u + jnp.uint32(0x7FFF) + ((u >> 16) & jnp.uint32(1))) >> 16
    packed = (r[:, _H // 2:] << 16) | r[:, :_H // 2]
    return lax.bitcast_convert_type(packed, jnp.int32)


def _k_pre(x, Wn, bn, Wc, Wsd):
    def body(x_ref, wn_ref, bn_ref, wc_ref, wsd_ref, xp_ref, asad_ref):
        h = jnp.dot(x_ref[...], wn_ref[...], preferred_element_type=_f32)
        h = h + bn_ref[...]
        xp_ref[...] = _pack_xp(h, wc_ref[...])
        asad_ref[...] = jnp.dot(h, wsd_ref[...], preferred_element_type=_f32)

    return pl.pallas_call(
        body,
        grid=(_N // _BN,),
        in_specs=[
            pl.BlockSpec((_BN, _H), lambda i: (i, 0)),
            pl.BlockSpec((_H, _H), lambda i: (0, 0)),
            pl.BlockSpec((1, _H), lambda i: (0, 0)),
            pl.BlockSpec((_H, _H), lambda i: (0, 0)),
            pl.BlockSpec((_H, 16), lambda i: (0, 0)),
        ],
        out_specs=[
            pl.BlockSpec((_BN, _H // 2), lambda i: (i, 0)),
            pl.BlockSpec((_BN, 16), lambda i: (i, 0)),
        ],
        out_shape=[
            jax.ShapeDtypeStruct((_N, _H // 2), jnp.int32),
            jax.ShapeDtypeStruct((_N, 16), _f32),
        ],
    )(x, Wn, bn, Wc, Wsd)


def _k_edge(ea16, W2, bae128):
    EP = _E // 16
    BR = 2000

    def body(ea_ref, w_ref, b_ref, out_ref):
        out_ref[0] = jnp.dot(ea_ref[...], w_ref[0],
                             preferred_element_type=_f32) + b_ref[0]

    return pl.pallas_call(
        body,
        grid=(_L, EP // BR),
        in_specs=[
            pl.BlockSpec((BR, 256), lambda l, i: (i, 0)),
            pl.BlockSpec((1, 256, 128), lambda l, i: (l, 0, 0)),
            pl.BlockSpec((1, 1, 128), lambda l, i: (l, 0, 0)),
        ],
        out_specs=pl.BlockSpec((1, BR, 128), lambda l, i: (l, i, 0)),
        out_shape=jax.ShapeDtypeStruct((_L, EP, 128), _f32),
    )(ea16, W2, bae128)


def _k_post(pm, pd, R, bc, g, b, h_in, Wc2, Wsd2, has_res, has_next):
    def body(*refs):
        if has_res:
            (pm_ref, pd_ref, r_ref, bc_ref, g_ref, b_ref, hin_ref) = refs[:7]
            rest = refs[7:]
        else:
            (pm_ref, pd_ref, r_ref, bc_ref, g_ref, b_ref) = refs[:6]
            rest = refs[6:]
        if has_next:
            wc_ref, wsd_ref = rest[:2]
            out_refs = rest[2:]
        else:
            out_refs = rest

        m = pm_ref[0] + pm_ref[1]
        d = pd_ref[0] + pd_ref[1]
        dexp = jnp.dot(d, r_ref[...], preferred_element_type=_f32)
        t = m / (dexp + 1e-16) + bc_ref[...]
        mu = jnp.mean(t, axis=-1, keepdims=True)
        tc = t - mu
        var = jnp.mean(tc * tc, axis=-1, keepdims=True)
        t = g_ref[...] * tc * lax.rsqrt(var + 1e-5) + b_ref[...]
        t = _gelu(t)
        if has_res:
            t = t + hin_ref[...]
        out_refs[0][...] = t
        if has_next:
            out_refs[1][...] = _pack_xp(t, wc_ref[...])
            out_refs[2][...] = jnp.dot(t, wsd_ref[...], preferred_element_type=_f32)

    in_specs = [
        pl.BlockSpec((_NC, _BN, _H), lambda i: (0, i, 0)),
        pl.BlockSpec((_NC, _BN, 16), lambda i: (0, i, 0)),
        pl.BlockSpec((16, _H), lambda i: (0, 0)),
        pl.BlockSpec((1, _H), lambda i: (0, 0)),
        pl.BlockSpec((1, _H), lambda i: (0, 0)),
        pl.BlockSpec((1, _H), lambda i: (0, 0)),
    ]
    args = [pm, pd, R, bc, g, b]
    if has_res:
        in_specs.append(pl.BlockSpec((_BN, _H), lambda i: (i, 0)))
        args.append(h_in)
    out_specs = [pl.BlockSpec((_BN, _H), lambda i: (i, 0))]
    out_shape = [jax.ShapeDtypeStruct((_N, _H), _f32)]
    if has_next:
        in_specs.append(pl.BlockSpec((_H, _H), lambda i: (0, 0)))
        in_specs.append(pl.BlockSpec((_H, 16), lambda i: (0, 0)))
        args.append(Wc2)
        args.append(Wsd2)
        out_specs.append(pl.BlockSpec((_BN, _H // 2), lambda i: (i, 0)))
        out_specs.append(pl.BlockSpec((_BN, 16), lambda i: (i, 0)))
        out_shape.append(jax.ShapeDtypeStruct((_N, _H // 2), jnp.int32))
        out_shape.append(jax.ShapeDtypeStruct((_N, 16), _f32))

    res = pl.pallas_call(
        body,
        grid=(_N // _BN,),
        in_specs=in_specs,
        out_specs=out_specs,
        out_shape=out_shape,
    )(*args)
    return res if has_next else (res[0], None, None)


def _k_pool(h, batch3, Wp1, bp1, Wp2, bp2, Wp3, bp3, Wm1, bm1, Wm2, bm2, Wm3, bm3):
    nblk = _N // _BN

    def body(h_ref, b_ref, wp1, bp1r, wp2, bp2r, wp3, bp3r,
             wm1, bm1r, wm2, bm2r, wm3, bm3r, po_ref, mo_ref, sums, cnts):
        i = pl.program_id(0)

        @pl.when(i == 0)
        def _():
            sums[...] = jnp.zeros_like(sums)
            cnts[...] = jnp.zeros_like(cnts)

        bb = b_ref[0]
        gi = lax.broadcasted_iota(jnp.int32, (_G, _BN), 0)
        oh = (gi == jnp.broadcast_to(bb, (_G, _BN))).astype(_f32)
        sums[...] += jnp.dot(oh, h_ref[...], preferred_element_type=_f32)
        cnts[...] += jnp.broadcast_to(
            jnp.sum(oh, axis=1, keepdims=True), (_G, _H))

        @pl.when(i == nblk - 1)
        def _():
            sm = sums[...]
            mean = sm / jnp.maximum(cnts[...], 1.0)
            gfeat = jnp.concatenate([mean, sm], axis=1)

            def mlp(w1, b1, w2, b2, w3, b3):
                h1 = _gelu(jnp.dot(gfeat, w1[...], preferred_element_type=_f32) + b1[...])
                h2 = _gelu(jnp.dot(h1, w2[...], preferred_element_type=_f32) + b2[...])
                o = jnp.dot(h2, w3[...], preferred_element_type=_f32) + b3[...]
                return 1.0 / (1.0 + jnp.exp(-o))

            po_ref[...] = mlp(wp1, bp1r, wp2, bp2r, wp3, bp3r)
            mo_ref[...] = mlp(wm1, bm1r, wm2, bm2r, wm3, bm3r)

    wspec = lambda shape: pl.BlockSpec(shape, lambda i: tuple(0 for _ in shape))
    return pl.pallas_call(
        body,
        grid=(nblk,),
        in_specs=[
            pl.BlockSpec((_BN, _H), lambda i: (i, 0)),
            pl.BlockSpec((1, 1, _BN), lambda i: (i, 0, 0)),
            wspec((2 * _H, _H)), wspec((1, _H)),
            wspec((_H, _H // 2)), wspec((1, _H // 2)),
            wspec((_H // 2, _H)), wspec((1, _H)),
            wspec((2 * _H, _H)), wspec((1, _H)),
            wspec((_H, _H // 2)), wspec((1, _H // 2)),
            wspec((_H // 2, _H)), wspec((1, _H)),
        ],
        out_specs=[
            pl.BlockSpec((_G, _H), lambda i: (0, 0)),
            pl.BlockSpec((_G, _H), lambda i: (0, 0)),
        ],
        out_shape=[
            jax.ShapeDtypeStruct((_G, _H), _f32),
            jax.ShapeDtypeStruct((_G, _H), _f32),
        ],
        scratch_shapes=[
            pltpu.VMEM((_G, _H), _f32),
            pltpu.VMEM((_G, _H), _f32),
        ],
    )(h, batch3, Wp1, bp1, Wp2, bp2, Wp3, bp3, Wm1, bm1, Wm2, bm2, Wm3, bm3)


def kernel(x, edge_index, edge_attr, batch, W_node, b_node, W_edge, b_edge,
           W_conv, att_src, att_dst, W_cedge, att_edge, b_conv, ln_g, ln_b,
           W_p1, b_p1, W_p2, b_p2, W_p3, b_p3, W_m1, b_m1, W_m2, b_m2,
           W_m3, b_m3):
    eye = jnp.eye(_NH, dtype=_f32)
    A_s = (att_src[:, :, :, None] * eye[None, :, None, :]).reshape(_L, _H, _NH)
    A_d = (att_dst[:, :, :, None] * eye[None, :, None, :]).reshape(_L, _H, _NH)
    A_e = (att_edge[:, :, :, None] * eye[None, :, None, :]).reshape(_L, _H, _NH)
    Bs = jnp.einsum("lij,ljk->lik", W_conv, A_s)
    Bd = jnp.einsum("lij,ljk->lik", W_conv, A_d)
    Wsd = jnp.concatenate([Bs, Bd], axis=-1)
    Me = jnp.einsum("lij,ljk->lik", W_cedge, A_e)
    WaeL = jnp.einsum("di,lik->ldk", W_edge, Me)
    W2 = jnp.einsum("ldh,jJ->ljdJh", WaeL,
                    jnp.eye(16, dtype=_f32)).reshape(_L, 256, 128)
    baeL = jnp.einsum("i,lik->lk", b_edge, Me)
    bae128 = jnp.tile(baeL, (1, 16)).reshape(_L, 1, 128)

    R = (eye[:, :, None] * jnp.ones((1, 1, 16), _f32)).reshape(_NH, _H)
    R = jnp.concatenate([R, jnp.zeros((8, _H), _f32)], axis=0)

    jq = jnp.arange(_H)
    jr = jq % 64
    srcq = 32 * (jr // 16) + jnp.where(jq >= 64, 16, 0) + (jr % 16)
    Q = jnp.zeros((_H, _H), _f32).at[srcq, jq].set(1.0)
    WcP = jnp.einsum("lij,jk->lik", W_conv, Q)

    bn2 = b_node.reshape(1, _H)
    src2 = edge_index[0].reshape(_NCHUNK, _CH)
    dst2 = edge_index[1].reshape(_NCHUNK, _CH)
    zm = jnp.zeros((_NPAD, _H), _f32)
    zd = jnp.zeros((_NPAD, 16), _f32)
    batch3 = batch.reshape(_N // _BN, 1, _BN)

    ae_all = _k_edge(edge_attr.reshape(_E // 16, 256), W2, bae128)

    xp, asad = _k_pre(x, W_node, bn2, WcP[0], Wsd[0])

    pad = lambda w, b: (
        jnp.concatenate([w, jnp.zeros((w.shape[0], _H - w.shape[1]), _f32)], 1),
        jnp.concatenate([b, jnp.zeros((_H - b.shape[0],), _f32)]).reshape(1, _H),
    )
    Wp3p, bp3p = pad(W_p3, b_p3)
    Wm3p, bm3p = pad(W_m3, b_m3)

    h = None
    for i in range(_L):
        pm, pd = _sc_edge_call(xp, asad, ae_all, i, src2, dst2, zm, zd)
        has_next = i < _L - 1
        h, xp, asad = _k_post(
            pm, pd, R, b_conv[i].reshape(1, _H), ln_g[i].reshape(1, _H),
            ln_b[i].reshape(1, _H), h,
            WcP[i + 1] if has_next else None,
            Wsd[i + 1] if has_next else None,
            has_res=(i > 0), has_next=has_next)

    params, metrics = _k_pool(
        h, batch3,
        W_p1, b_p1.reshape(1, _H), W_p2, b_p2.reshape(1, _H // 2), Wp3p, bp3p,
        W_m1, b_m1.reshape(1, _H), W_m2, b_m2.reshape(1, _H // 2), Wm3p, bm3p)
    return (params[:, :3], metrics[:, :3])


# speedup vs baseline: 1.4045x; 1.4045x over previous
import functools

import jax
import jax.numpy as jnp
from jax import lax
from jax.experimental import pallas as pl
from jax.experimental.pallas import tpu as pltpu
from jax.experimental.pallas import tpu_sc as plsc

_N = 10000
_E = 320000
_H = 128
_NH = 8
_L = 6
_G = 8

_CH = 64
_NCHUNK = _E // _CH
_NC = 2
_NS = 16
_NW = _NC * _NS
_NPAD = 10240
_NPS = _NPAD // _NS

_f32 = jnp.float32


def _sc_edge_call(xp, asad, ae, li, src2, dst2, zm, zd):
    mesh = plsc.VectorSubcoreMesh(core_axis_name="c", subcore_axis_name="s")

    @functools.partial(
        pl.kernel,
        out_type=(
            jax.ShapeDtypeStruct((_NC, _NPAD, _H), _f32),
            jax.ShapeDtypeStruct((_NC, _NPAD, 16), _f32),
        ),
        mesh=mesh,
        compiler_params=pltpu.CompilerParams(use_tc_tiling_on_sc=False),
        scratch_types=[
            pltpu.VMEM_SHARED((_NPAD, _H), _f32),
            pltpu.VMEM_SHARED((_NPAD, 16), _f32),
            pltpu.VMEM((3, _CH), jnp.int32),
            pltpu.VMEM((6, _CH), jnp.int32),
            pltpu.VMEM((2, _CH, 16), _f32),
            pltpu.VMEM((2, _CH, 16), _f32),
            pltpu.VMEM((2, _CH // 16, 128), _f32),
            pltpu.VMEM((3, _CH, _H), _f32),
            pltpu.VMEM((3, _CH, 16), _f32),
        ] + [pltpu.SemaphoreType.DMA] * 18,
    )
    def k(xp_h, asad_h, ae_h, src_h, dst_h, zm_h, zd_h, om_h, od_h,
          accm, accd, srcv, dstv, asv, adv, aev, xpv, exv, *sems):
        s_as = sems[0:2]
        s_ad = sems[2:4]
        s_ae = sems[4:6]
        s_xp = sems[6:9]
        s_sm = sems[9:12]
        s_sd = sems[12:15]
        s_ix = sems[15:18]
        c = lax.axis_index("c")
        s = lax.axis_index("s")
        w = c * _NS + s

        r0 = s * _NPS
        pltpu.sync_copy(zm_h.at[pl.ds(r0, _NPS)], accm.at[pl.ds(r0, _NPS)])
        pltpu.sync_copy(zd_h.at[pl.ds(r0, _NPS)], accd.at[pl.ds(r0, _NPS)])
        plsc.subcore_barrier()

        ii = lax.iota(jnp.int32, 16)
        perm8 = ii ^ 8

        nfull = _NCHUNK // _NW
        rem = _NCHUNK - nfull * _NW
        n_my = nfull + jnp.where(w < rem, 1, 0)

        def issue_idx(kk, i3, i6):
            ci = w + kk * _NW
            pltpu.async_copy(src_h.at[ci], srcv.at[i3], s_ix[i3])
            pltpu.async_copy(dst_h.at[ci], dstv.at[i6], s_ix[i3])

        def wait_idx(i3, i6):
            pltpu.make_async_copy(src_h.at[0], srcv.at[i3], s_ix[i3]).wait()
            pltpu.make_async_copy(dst_h.at[0], dstv.at[i6], s_ix[i3]).wait()

        def issue_gathers(kk, i2, i3, i6):
            ci = w + kk * _NW
            pltpu.async_copy(asad_h.at[srcv.at[i3]], asv.at[i2], s_as[i2])
            pltpu.async_copy(asad_h.at[dstv.at[i6]], adv.at[i2], s_ad[i2])
            pltpu.async_copy(xp_h.at[srcv.at[i3]], xpv.at[i3], s_xp[i3])
            pltpu.async_copy(
                ae_h.at[li, pl.ds(ci * (_CH // 16), _CH // 16)],
                aev.at[i2], s_ae[i2])

        def wait_gathers(i2, i3):
            pltpu.make_async_copy(asad_h.at[srcv.at[i3]], asv.at[i2],
                                  s_as[i2]).wait()
            pltpu.make_async_copy(asad_h.at[srcv.at[i3]], adv.at[i2],
                                  s_ad[i2]).wait()
            pltpu.make_async_copy(xp_h.at[srcv.at[i3]], xpv.at[i3],
                                  s_xp[i3]).wait()
            pltpu.make_async_copy(ae_h.at[0, pl.ds(0, _CH // 16)],
                                  aev.at[i2], s_ae[i2]).wait()

        def issue_scatters(i3, i6):
            pltpu.async_copy(xpv.at[i3], accm.at[dstv.at[i6]], s_sm[i3],
                             add=True)
            pltpu.async_copy(exv.at[i3], accd.at[dstv.at[i6]], s_sd[i3],
                             add=True)

        def wait_scatters(i3):
            pltpu.make_async_copy(xpv.at[i3], accm.at[dstv.at[0]],
                                  s_sm[i3]).wait()
            pltpu.make_async_copy(exv.at[i3], accd.at[dstv.at[0]],
                                  s_sd[i3]).wait()

        def compute(i2, i3):
            def edge_body(j, carry2):
                aerow = aev[i2, j // 8, pl.ds((j % 8) * 16, 16)]
                for t in range(2):
                    e = j * 2 + t
                    va = asv[i2, e]
                    vd = adv[i2, e]
                    vae = aerow if t == 0 else jnp.take(aerow, perm8)
                    alpha = va + jnp.take(vd, perm8) + vae
                    alpha = jnp.where(alpha > 0, alpha, alpha * 0.2)
                    ex = jnp.exp(alpha)
                    exv[i3, e] = ex
                    for h in range(_NH):
                        sl = pl.ds(16 * h, 16)
                        xrow = xpv[i3, e, sl]
                        exb = jnp.take(ex, jnp.full((16,), h, jnp.int32))
                        xpv[i3, e, sl] = xrow * exb
                return carry2

            lax.fori_loop(0, _CH // 2, edge_body, 0)

        issue_idx(0, 0, 0)
        issue_idx(1, 1, 1)
        wait_idx(0, 0)
        issue_gathers(0, 0, 0, 0)

        def step_body(g, carry):
            for bb in range(6):
                kk = g * 6 + bb
                u2 = bb % 2
                u3 = bb % 3
                u6 = bb
                q2 = (bb + 1) % 2
                q3 = (bb + 1) % 3
                q6 = (bb + 1) % 6

                @pl.when(kk < n_my)
                def _():
                    @pl.when(kk + 2 < n_my)
                    def _():
                        issue_idx(kk + 2, (bb + 2) % 3, (bb + 2) % 6)

                    wait_gathers(u2, u3)

                    @pl.when(kk + 1 < n_my)
                    def _():
                        @pl.when(kk >= 2)
                        def _():
                            wait_scatters(q3)
                        wait_idx(q3, q6)
                        issue_gathers(kk + 1, q2, q3, q6)

                    compute(u2, u3)
                    issue_scatters(u3, u6)
            return carry

        lax.fori_loop(0, (n_my + 5) // 6, step_body, 0)
        wait_scatters(0)
        wait_scatters(1)
        wait_scatters(2)

        plsc.subcore_barrier()
        pltpu.sync_copy(accm.at[pl.ds(r0, _NPS)], om_h.at[c, pl.ds(r0, _NPS)])
        pltpu.sync_copy(accd.at[pl.ds(r0, _NPS)], od_h.at[c, pl.ds(r0, _NPS)])

    return k(xp, asad, ae, src2, dst2, zm, zd)


_BN = 1000


def _gelu(t):
    return t * 0.5 * (1.0 + lax.erf(t * 0.7071067811865476))


def _k_pre(x, Wn, bn, Wc, Wsd):
    def body(x_ref, wn_ref, bn_ref, wc_ref, wsd_ref, xp_ref, asad_ref):
        h = jnp.dot(x_ref[...], wn_ref[...], preferred_element_type=_f32)
        h = h + bn_ref[...]
        xp_ref[...] = jnp.dot(h, wc_ref[...], preferred_element_type=_f32)
        asad_ref[...] = jnp.dot(h, wsd_ref[...], preferred_element_type=_f32)

    return pl.pallas_call(
        body,
        grid=(_N // _BN,),
        in_specs=[
            pl.BlockSpec((_BN, _H), lambda i: (i, 0)),
            pl.BlockSpec((_H, _H), lambda i: (0, 0)),
            pl.BlockSpec((1, _H), lambda i: (0, 0)),
            pl.BlockSpec((_H, _H), lambda i: (0, 0)),
            pl.BlockSpec((_H, 16), lambda i: (0, 0)),
        ],
        out_specs=[
            pl.BlockSpec((_BN, _H), lambda i: (i, 0)),
            pl.BlockSpec((_BN, 16), lambda i: (i, 0)),
        ],
        out_shape=[
            jax.ShapeDtypeStruct((_N, _H), _f32),
            jax.ShapeDtypeStruct((_N, 16), _f32),
        ],
    )(x, Wn, bn, Wc, Wsd)


def _k_edge(ea16, W2, bae128):
    EP = _E // 16
    BR = 2000

    def body(ea_ref, w_ref, b_ref, out_ref):
        out_ref[0] = jnp.dot(ea_ref[...], w_ref[0],
                             preferred_element_type=_f32) + b_ref[0]

    return pl.pallas_call(
        body,
        grid=(_L, EP // BR),
        in_specs=[
            pl.BlockSpec((BR, 256), lambda l, i: (i, 0)),
            pl.BlockSpec((1, 256, 128), lambda l, i: (l, 0, 0)),
            pl.BlockSpec((1, 1, 128), lambda l, i: (l, 0, 0)),
        ],
        out_specs=pl.BlockSpec((1, BR, 128), lambda l, i: (l, i, 0)),
        out_shape=jax.ShapeDtypeStruct((_L, EP, 128), _f32),
    )(ea16, W2, bae128)


def _k_post(pm, pd, R, bc, g, b, h_in, Wc2, Wsd2, has_res, has_next):
    def body(*refs):
        if has_res:
            (pm_ref, pd_ref, r_ref, bc_ref, g_ref, b_ref, hin_ref) = refs[:7]
            rest = refs[7:]
        else:
            (pm_ref, pd_ref, r_ref, bc_ref, g_ref, b_ref) = refs[:6]
            rest = refs[6:]
        if has_next:
            wc_ref, wsd_ref = rest[:2]
            out_refs = rest[2:]
        else:
            out_refs = rest

        m = pm_ref[0] + pm_ref[1]
        d = pd_ref[0] + pd_ref[1]
        dexp = jnp.dot(d, r_ref[...], preferred_element_type=_f32)
        t = m / (dexp + 1e-16) + bc_ref[...]
        mu = jnp.mean(t, axis=-1, keepdims=True)
        tc = t - mu
        var = jnp.mean(tc * tc, axis=-1, keepdims=True)
        t = g_ref[...] * tc * lax.rsqrt(var + 1e-5) + b_ref[...]
        t = _gelu(t)
        if has_res:
            t = t + hin_ref[...]
        out_refs[0][...] = t
        if has_next:
            out_refs[1][...] = jnp.dot(t, wc_ref[...], preferred_element_type=_f32)
            out_refs[2][...] = jnp.dot(t, wsd_ref[...], preferred_element_type=_f32)

    in_specs = [
        pl.BlockSpec((_NC, _BN, _H), lambda i: (0, i, 0)),
        pl.BlockSpec((_NC, _BN, 16), lambda i: (0, i, 0)),
        pl.BlockSpec((16, _H), lambda i: (0, 0)),
        pl.BlockSpec((1, _H), lambda i: (0, 0)),
        pl.BlockSpec((1, _H), lambda i: (0, 0)),
        pl.BlockSpec((1, _H), lambda i: (0, 0)),
    ]
    args = [pm, pd, R, bc, g, b]
    if has_res:
        in_specs.append(pl.BlockSpec((_BN, _H), lambda i: (i, 0)))
        args.append(h_in)
    out_specs = [pl.BlockSpec((_BN, _H), lambda i: (i, 0))]
    out_shape = [jax.ShapeDtypeStruct((_N, _H), _f32)]
    if has_next:
        in_specs.append(pl.BlockSpec((_H, _H), lambda i: (0, 0)))
        in_specs.append(pl.BlockSpec((_H, 16), lambda i: (0, 0)))
        args.append(Wc2)
        args.append(Wsd2)
        out_specs.append(pl.BlockSpec((_BN, _H), lambda i: (i, 0)))
        out_specs.append(pl.BlockSpec((_BN, 16), lambda i: (i, 0)))
        out_shape.append(jax.ShapeDtypeStruct((_N, _H), _f32))
        out_shape.append(jax.ShapeDtypeStruct((_N, 16), _f32))

    res = pl.pallas_call(
        body,
        grid=(_N // _BN,),
        in_specs=in_specs,
        out_specs=out_specs,
        out_shape=out_shape,
    )(*args)
    return res if has_next else (res[0], None, None)


def _k_pool(h, batch3, Wp1, bp1, Wp2, bp2, Wp3, bp3, Wm1, bm1, Wm2, bm2, Wm3, bm3):
    nblk = _N // _BN

    def body(h_ref, b_ref, wp1, bp1r, wp2, bp2r, wp3, bp3r,
             wm1, bm1r, wm2, bm2r, wm3, bm3r, po_ref, mo_ref, sums, cnts):
        i = pl.program_id(0)

        @pl.when(i == 0)
        def _():
            sums[...] = jnp.zeros_like(sums)
            cnts[...] = jnp.zeros_like(cnts)

        bb = b_ref[0]
        gi = lax.broadcasted_iota(jnp.int32, (_G, _BN), 0)
        oh = (gi == jnp.broadcast_to(bb, (_G, _BN))).astype(_f32)
        sums[...] += jnp.dot(oh, h_ref[...], preferred_element_type=_f32)
        cnts[...] += jnp.broadcast_to(
            jnp.sum(oh, axis=1, keepdims=True), (_G, _H))

        @pl.when(i == nblk - 1)
        def _():
            sm = sums[...]
            mean = sm / jnp.maximum(cnts[...], 1.0)
            gfeat = jnp.concatenate([mean, sm], axis=1)

            def mlp(w1, b1, w2, b2, w3, b3):
                h1 = _gelu(jnp.dot(gfeat, w1[...], preferred_element_type=_f32) + b1[...])
                h2 = _gelu(jnp.dot(h1, w2[...], preferred_element_type=_f32) + b2[...])
                o = jnp.dot(h2, w3[...], preferred_element_type=_f32) + b3[...]
                return 1.0 / (1.0 + jnp.exp(-o))

            po_ref[...] = mlp(wp1, bp1r, wp2, bp2r, wp3, bp3r)
            mo_ref[...] = mlp(wm1, bm1r, wm2, bm2r, wm3, bm3r)

    wspec = lambda shape: pl.BlockSpec(shape, lambda i: tuple(0 for _ in shape))
    return pl.pallas_call(
        body,
        grid=(nblk,),
        in_specs=[
            pl.BlockSpec((_BN, _H), lambda i: (i, 0)),
            pl.BlockSpec((1, 1, _BN), lambda i: (i, 0, 0)),
            wspec((2 * _H, _H)), wspec((1, _H)),
            wspec((_H, _H // 2)), wspec((1, _H // 2)),
            wspec((_H // 2, _H)), wspec((1, _H)),
            wspec((2 * _H, _H)), wspec((1, _H)),
            wspec((_H, _H // 2)), wspec((1, _H // 2)),
            wspec((_H // 2, _H)), wspec((1, _H)),
        ],
        out_specs=[
            pl.BlockSpec((_G, _H), lambda i: (0, 0)),
            pl.BlockSpec((_G, _H), lambda i: (0, 0)),
        ],
        out_shape=[
            jax.ShapeDtypeStruct((_G, _H), _f32),
            jax.ShapeDtypeStruct((_G, _H), _f32),
        ],
        scratch_shapes=[
            pltpu.VMEM((_G, _H), _f32),
            pltpu.VMEM((_G, _H), _f32),
        ],
    )(h, batch3, Wp1, bp1, Wp2, bp2, Wp3, bp3, Wm1, bm1, Wm2, bm2, Wm3, bm3)


def kernel(x, edge_index, edge_attr, batch, W_node, b_node, W_edge, b_edge,
           W_conv, att_src, att_dst, W_cedge, att_edge, b_conv, ln_g, ln_b,
           W_p1, b_p1, W_p2, b_p2, W_p3, b_p3, W_m1, b_m1, W_m2, b_m2,
           W_m3, b_m3):
    eye = jnp.eye(_NH, dtype=_f32)
    A_s = (att_src[:, :, :, None] * eye[None, :, None, :]).reshape(_L, _H, _NH)
    A_d = (att_dst[:, :, :, None] * eye[None, :, None, :]).reshape(_L, _H, _NH)
    A_e = (att_edge[:, :, :, None] * eye[None, :, None, :]).reshape(_L, _H, _NH)
    Bs = jnp.einsum("lij,ljk->lik", W_conv, A_s)
    Bd = jnp.einsum("lij,ljk->lik", W_conv, A_d)
    Wsd = jnp.concatenate([Bs, Bd], axis=-1)
    Me = jnp.einsum("lij,ljk->lik", W_cedge, A_e)
    WaeL = jnp.einsum("di,lik->ldk", W_edge, Me)
    W2 = jnp.einsum("ldh,jJ->ljdJh", WaeL,
                    jnp.eye(16, dtype=_f32)).reshape(_L, 256, 128)
    baeL = jnp.einsum("i,lik->lk", b_edge, Me)
    bae128 = jnp.tile(baeL, (1, 16)).reshape(_L, 1, 128)

    R = (eye[:, :, None] * jnp.ones((1, 1, 16), _f32)).reshape(_NH, _H)
    R = jnp.concatenate([R, jnp.zeros((8, _H), _f32)], axis=0)

    bn2 = b_node.reshape(1, _H)
    src2 = edge_index[0].reshape(_NCHUNK, _CH)
    dst2 = edge_index[1].reshape(_NCHUNK, _CH)
    zm = jnp.zeros((_NPAD, _H), _f32)
    zd = jnp.zeros((_NPAD, 16), _f32)
    batch3 = batch.reshape(_N // _BN, 1, _BN)

    ae_all = _k_edge(edge_attr.reshape(_E // 16, 256), W2, bae128)

    xp, asad = _k_pre(x, W_node, bn2, W_conv[0], Wsd[0])

    pad = lambda w, b: (
        jnp.concatenate([w, jnp.zeros((w.shape[0], _H - w.shape[1]), _f32)], 1),
        jnp.concatenate([b, jnp.zeros((_H - b.shape[0],), _f32)]).reshape(1, _H),
    )
    Wp3p, bp3p = pad(W_p3, b_p3)
    Wm3p, bm3p = pad(W_m3, b_m3)

    h = None
    for i in range(_L):
        pm, pd = _sc_edge_call(xp, asad, ae_all, i, src2, dst2, zm, zd)
        has_next = i < _L - 1
        h, xp, asad = _k_post(
            pm, pd, R, b_conv[i].reshape(1, _H), ln_g[i].reshape(1, _H),
            ln_b[i].reshape(1, _H), h,
            W_conv[i + 1] if has_next else None,
            Wsd[i + 1] if has_next else None,
            has_res=(i > 0), has_next=has_next)

    params, metrics = _k_pool(
        h, batch3,
        W_p1, b_p1.reshape(1, _H), W_p2, b_p2.reshape(1, _H // 2), Wp3p, bp3p,
        W_m1, b_m1.reshape(1, _H), W_m2, b_m2.reshape(1, _H // 2), Wm3p, bm3p)
    return (params[:, :3], metrics[:, :3])
